# Initial kernel scaffold; baseline (speedup 1.0000x reference)
import jax
import jax.numpy as jnp
from jax.experimental import pallas as pl


def kernel(x, edge_index, edge_weight, W1, b1, W2, b2, eps, alpha, gamma, beta):
    raise NotImplementedError("write your pallas kernel here")


# speedup vs baseline: 2.8575x; 2.8575x over previous
import functools

import jax
import jax.numpy as jnp
from jax import lax
from jax.experimental import pallas as pl
from jax.experimental.pallas import tpu as pltpu
from jax.experimental.pallas import tpu_sc as plsc

N = 10000
E = 160000
D = 256
DH = 128
NS = 16
NC = 2
CHUNK = 128
NCHUNK_PER_TILE = 80
NSTAGE = 2
NCHUNK_STAGE = NCHUNK_PER_TILE // NSTAGE
EPS = CHUNK * NCHUNK_STAGE
EPT = CHUNK * NCHUNK_PER_TILE
EPAD = EPT * NS
NPAD = 10240
NPT = NPAD // NS


def _sc_body(xflat, src_h, dst_h, w_h, out_h,
             gidxv, dstv, wv, rows, acc, sem0, sem1):
    c = lax.axis_index("c")
    s = lax.axis_index("s")

    zv = jnp.zeros((16,), jnp.float32)

    def zbuf(i, carry):
        r = i // 8
        k = i % 8
        rows[0, r, pl.ds(k * 16, 16)] = zv
        return carry

    lax.fori_loop(0, 128 * 8, zbuf, 0)

    for q in range(NPT // 128):
        pltpu.sync_copy(rows.at[0],
                        acc.at[pl.ds(s * NPT + q * 128, 128)])

    plsc.subcore_barrier()

    cvec = jnp.full((16,), c, dtype=jnp.int32)

    def gather(j, slot, sem):
        return pltpu.async_copy(
            xflat.at[gidxv.at[pl.ds(j * CHUNK, CHUNK)]], rows.at[slot], sem)

    def process(j, slot):
        def pgroup(g, carry):
            wvec16 = wv[j, pl.ds(g * 16, 16)]
            for rr in range(16):
                wb = jnp.full((16,), wvec16[rr])
                r = g * 16 + rr
                for k in range(8):
                    rows[slot, r, pl.ds(k * 16, 16)] = (
                        rows[slot, r, pl.ds(k * 16, 16)] * wb)
            return carry

        lax.fori_loop(0, CHUNK // 16, pgroup, 0)
        pltpu.sync_copy(rows.at[slot], acc.at[dstv.at[j]], add=True)

    for h in range(NSTAGE):
        pltpu.sync_copy(src_h.at[s, h], gidxv)
        pltpu.sync_copy(dst_h.at[s, h], dstv)
        pltpu.sync_copy(w_h.at[s, h], wv)

        def gix(i, carry):
            v = gidxv[pl.ds(i * 16, 16)]
            gidxv[pl.ds(i * 16, 16)] = v * 2 + cvec
            return carry

        lax.fori_loop(0, EPS // 16, gix, 0)

        def pair(i, carry):
            j0 = i * 2
            gather(j0, 0, sem0)
            gather(j0 + 1, 1, sem1)
            pltpu.make_async_copy(xflat.at[gidxv.at[pl.ds(0, CHUNK)]],
                                  rows.at[0], sem0).wait()
            process(j0, 0)
            pltpu.make_async_copy(xflat.at[gidxv.at[pl.ds(0, CHUNK)]],
                                  rows.at[1], sem1).wait()
            process(j0 + 1, 1)
            return carry

        lax.fori_loop(0, NCHUNK_STAGE // 2, pair, 0)

    plsc.subcore_barrier()
    pltpu.sync_copy(acc.at[pl.ds(s * NPT, NPT)],
                    out_h.at[c, pl.ds(s * NPT, NPT)])


def _sc_scatter(xflat, src3, dst3, w3):
    mesh = plsc.VectorSubcoreMesh(core_axis_name="c", subcore_axis_name="s")
    return pl.kernel(
        _sc_body,
        out_type=jax.ShapeDtypeStruct((NC, NPAD, DH), jnp.float32),
        mesh=mesh,
        scratch_types=[
            pltpu.VMEM((EPS,), jnp.int32),
            pltpu.VMEM((NCHUNK_STAGE, CHUNK), jnp.int32),
            pltpu.VMEM((NCHUNK_STAGE, CHUNK), jnp.float32),
            pltpu.VMEM((2, CHUNK, DH), jnp.float32),
            pltpu.VMEM_SHARED((NPAD, DH), jnp.float32),
            pltpu.SemaphoreType.DMA,
            pltpu.SemaphoreType.DMA,
        ],
    )(xflat, src3, dst3, w3)


BN = 512


def _tc_body(x_ref, agg_ref, W1_ref, b1_ref, W2_ref, b2_ref,
             eps_ref, alpha_ref, gamma_ref, beta_ref, o_ref):
    x = x_ref[...]
    agg = jnp.concatenate([agg_ref[0], agg_ref[1]], axis=-1)
    out = agg + (1.0 + eps_ref[0, 0]) * x
    h = jnp.maximum(
        jnp.dot(out, W1_ref[...], preferred_element_type=jnp.float32)
        + b1_ref[...], 0.0)
    h = jnp.dot(h, W2_ref[...], preferred_element_type=jnp.float32) + b2_ref[...]
    h = jnp.maximum(h, 0.0)
    r = alpha_ref[...] * h + x
    ms = jnp.mean(r * r, axis=-1, keepdims=True)
    y = r / jnp.sqrt(ms + 1e-6)
    o_ref[...] = gamma_ref[...] * y + beta_ref[...]


def _tc_tail(x, aggsplit, W1, b1, W2, b2, eps, alpha, gamma, beta):
    grid = (pl.cdiv(N, BN),)
    full = lambda i: (0, 0)
    return pl.pallas_call(
        _tc_body,
        grid=grid,
        in_specs=[
            pl.BlockSpec((BN, D), lambda i: (i, 0)),
            pl.BlockSpec((NC, BN, DH), lambda i: (0, i, 0)),
            pl.BlockSpec((D, D), full),
            pl.BlockSpec((1, D), full),
            pl.BlockSpec((D, D), full),
            pl.BlockSpec((1, D), full),
            pl.BlockSpec((1, 1), full),
            pl.BlockSpec((1, D), full),
            pl.BlockSpec((1, D), full),
            pl.BlockSpec((1, D), full),
        ],
        out_specs=pl.BlockSpec((BN, D), lambda i: (i, 0)),
        out_shape=jax.ShapeDtypeStruct((N, D), jnp.float32),
    )(x, aggsplit, W1, b1.reshape(1, D), W2, b2.reshape(1, D),
      eps.reshape(1, 1), alpha.reshape(1, D), gamma.reshape(1, D),
      beta.reshape(1, D))


@jax.jit
def kernel(x, edge_index, edge_weight, W1, b1, W2, b2, eps, alpha, gamma, beta):
    src = edge_index[0].astype(jnp.int32)
    dst = edge_index[1].astype(jnp.int32)
    pad = EPAD - E
    src3 = jnp.concatenate([src, jnp.zeros((pad,), jnp.int32)]).reshape(
        NS, NSTAGE, EPS)
    dst3 = jnp.concatenate([dst, jnp.zeros((pad,), jnp.int32)]).reshape(
        NS, NSTAGE, NCHUNK_STAGE, CHUNK)
    w3 = jnp.concatenate(
        [edge_weight, jnp.zeros((pad,), jnp.float32)]).reshape(
        NS, NSTAGE, NCHUNK_STAGE, CHUNK)
    xflat = x.reshape(NC * N, DH)
    aggsplit = _sc_scatter(xflat, src3, dst3, w3)
    return _tc_tail(x, aggsplit, W1, b1, W2, b2, eps, alpha, gamma, beta)
